# Initial kernel scaffold; baseline (speedup 1.0000x reference)
import jax
import jax.numpy as jnp
from jax.experimental import pallas as pl


def kernel(x, edge_index, W0, as0, ad0, b0, g0, bt0, W1, as1, ad1, b1, g1, bt1, W2, as2, ad2, b2, g2, bt2, Wf, asf, adf, bf):
    raise NotImplementedError("write your pallas kernel here")


# speedup vs baseline: 16.8934x; 16.8934x over previous
import functools

import jax
import jax.numpy as jnp
from jax import lax
from jax.experimental import pallas as pl
from jax.experimental.pallas import tpu as pltpu
from jax.experimental.pallas import tpu_sc as plsc

N = 10000
DIN = 128
HID = 64
DOUT = 128
NEG = 0.2
E_REAL = 330000

NC = 2
NS = 16
NW = NC * NS
T = 10752
EPAD = NW * T
NPAD = 10240
RPT = NPAD // NS
CH = 512
NCHUNK = T // CH
NBUF = 2


def _make_sc_layer():
    D = HID
    mesh = plsc.VectorSubcoreMesh(core_axis_name="c", subcore_axis_name="s")

    scratch = dict(
        acc_sp=pltpu.VMEM_SHARED((NPAD, D), jnp.float32),
        den_sp=pltpu.VMEM_SHARED((NPAD,), jnp.float32),
        isem=pltpu.SemaphoreType.DMA,
        asem=pltpu.SemaphoreType.DMA,
        gsem=pltpu.SemaphoreType.DMA,
        dsem=pltpu.SemaphoreType.DMA,
        ssem=pltpu.SemaphoreType.DMA,
    )
    for b in range(NBUF):
        scratch[f"rows{b}"] = pltpu.VMEM((CH, D), jnp.float32)
        scratch[f"sidx{b}"] = pltpu.VMEM((CH,), jnp.int32)
        scratch[f"didx{b}"] = pltpu.VMEM((CH,), jnp.int32)
        scratch[f"av{b}"] = pltpu.VMEM((CH,), jnp.float32)
        scratch[f"bv{b}"] = pltpu.VMEM((CH,), jnp.float32)
        scratch[f"exc{b}"] = pltpu.VMEM((CH,), jnp.float32)

    @functools.partial(
        pl.kernel,
        out_type=(
            jax.ShapeDtypeStruct((NC, NPAD, D), jnp.float32),
            jax.ShapeDtypeStruct((NC, NPAD), jnp.float32),
        ),
        mesh=mesh,
        compiler_params=pltpu.CompilerParams(
            use_tc_tiling_on_sc=False, needs_layout_passes=False),
        scratch_types=scratch,
    )
    def sc_layer(xl_hbm, al_hbm, ar_hbm, srcw_hbm, dstw_hbm, zrows_hbm,
                 zden_hbm, acc_hbm, den_hbm, **scr):
        cid = lax.axis_index("c")
        sid = lax.axis_index("s")
        wid = cid * NS + sid
        rbase = sid * RPT

        rows = [scr[f"rows{b}"] for b in range(NBUF)]
        sidx = [scr[f"sidx{b}"] for b in range(NBUF)]
        didx = [scr[f"didx{b}"] for b in range(NBUF)]
        av = [scr[f"av{b}"] for b in range(NBUF)]
        bv = [scr[f"bv{b}"] for b in range(NBUF)]
        exc = [scr[f"exc{b}"] for b in range(NBUF)]
        acc_sp = scr["acc_sp"]
        den_sp = scr["den_sp"]
        isem, asem, gsem = scr["isem"], scr["asem"], scr["gsem"]
        dsem, ssem = scr["dsem"], scr["ssem"]

        pltpu.sync_copy(zrows_hbm, acc_sp.at[pl.ds(rbase, RPT)])
        pltpu.sync_copy(zden_hbm, den_sp.at[pl.ds(rbase, RPT)])
        plsc.subcore_barrier()

        iota16 = lax.iota(jnp.int32, 16)

        def start_chunk(c):
            b = c % NBUF
            i1 = pltpu.async_copy(
                srcw_hbm.at[wid].at[pl.ds(c * CH, CH)], sidx[b], isem)
            i2 = pltpu.async_copy(
                dstw_hbm.at[wid].at[pl.ds(c * CH, CH)], didx[b], isem)
            i1.wait()
            i2.wait()
            ha = pltpu.async_copy(al_hbm.at[sidx[b]], av[b], asem)
            hb = pltpu.async_copy(ar_hbm.at[didx[b]], bv[b], asem)
            hg = pltpu.async_copy(xl_hbm.at[sidx[b]], rows[b], gsem)
            return ha, hb, hg

        pend_d = [None] * NBUF
        pend_s = [None] * NBUF
        cur = start_chunk(0)
        for c in range(NCHUNK):
            b = c % NBUF
            ha, hb, hg = cur
            ha.wait()
            hb.wait()
            base_c = wid * T + c * CH

            def exp_body(j, carry, b=b, base_c=base_c):
                off = j * 16
                t = av[b][pl.ds(off, 16)] + bv[b][pl.ds(off, 16)]
                e = jnp.where(t >= 0.0, t, t * NEG)
                ex = jnp.exp(e)
                gid = base_c + off + iota16
                ex = jnp.where(gid < E_REAL, ex, 0.0)
                exc[b][pl.ds(off, 16)] = ex
                return carry

            lax.fori_loop(0, CH // 16, exp_body, 0)
            hd = pltpu.async_copy(exc[b], den_sp.at[didx[b]], dsem,
                                  add=True)
            hg.wait()

            def scale_body(jb, carry, b=b):
                exv = exc[b][pl.ds(jb * 16, 16)]
                for k in range(16):
                    e = exv[k]
                    row = jb * 16 + k
                    for g in range(D // 16):
                        sl = pl.ds(g * 16, 16)
                        rows[b][row, sl] = rows[b][row, sl] * e
                return carry

            lax.fori_loop(0, CH // 16, scale_body, 0)
            hs = pltpu.async_copy(rows[b], acc_sp.at[didx[b]], ssem,
                                  add=True)
            pend_d[b] = hd
            pend_s[b] = hs
            if c + 1 < NCHUNK:
                nb = (c + 1) % NBUF
                if pend_d[nb] is not None:
                    pend_d[nb].wait()
                    pend_s[nb].wait()
                cur = start_chunk(c + 1)
        for b in range(NBUF):
            if pend_d[b] is not None:
                pend_d[b].wait()
                pend_s[b].wait()

        plsc.subcore_barrier()
        pltpu.sync_copy(
            acc_sp.at[pl.ds(rbase, RPT)],
            acc_hbm.at[cid].at[pl.ds(rbase, RPT)])
        pltpu.sync_copy(
            den_sp.at[pl.ds(rbase, RPT)],
            den_hbm.at[cid].at[pl.ds(rbase, RPT)])

    return sc_layer


_sc_layer = _make_sc_layer()


_BN = 1000
_GRID = N // _BN


def _row_spec(width):
    return pl.BlockSpec((_BN, width), lambda i: (i, 0))


def _full_spec(shape):
    nd = len(shape)
    return pl.BlockSpec(shape, lambda i: (0,) * nd)


def _tc_in_body(x_ref, w_ref, as_ref, ad_ref, xl_ref, al_ref, ar_ref):
    xl = jnp.dot(x_ref[...], w_ref[...], preferred_element_type=jnp.float32)
    xl_ref[...] = xl
    al_ref[...] = jnp.sum(xl * as_ref[...], axis=-1, keepdims=True)
    ar_ref[...] = jnp.sum(xl * ad_ref[...], axis=-1, keepdims=True)


def _tc_in(x, w, a_s, a_d):
    din, h = w.shape
    return pl.pallas_call(
        _tc_in_body,
        grid=(_GRID,),
        in_specs=[
            _row_spec(din),
            _full_spec((din, h)),
            _full_spec((1, h)),
            _full_spec((1, h)),
        ],
        out_specs=[
            _row_spec(h),
            _row_spec(1),
            _row_spec(1),
        ],
        out_shape=[
            jax.ShapeDtypeStruct((N, h), jnp.float32),
            jax.ShapeDtypeStruct((N, 1), jnp.float32),
            jax.ShapeDtypeStruct((N, 1), jnp.float32),
        ],
    )(x, w, a_s, a_d)


def _tc_merge_body(acc0_ref, acc1_ref, den0_ref, den1_ref, b_ref, g_ref,
                   bt_ref, w_ref, as_ref, ad_ref, xl_ref, al_ref, ar_ref):
    a = acc0_ref[...] + acc1_ref[...]
    d = den0_ref[...] + den1_ref[...]
    h = a / (d + 1e-16) + b_ref[...]
    mu = jnp.mean(h, axis=-1, keepdims=True)
    var = jnp.mean((h - mu) ** 2, axis=-1, keepdims=True)
    h = (h - mu) / jnp.sqrt(var + 1e-5) * g_ref[...] + bt_ref[...]
    h = jnp.where(h > 0.0, h, jnp.exp(h) - 1.0)
    xl = jnp.dot(h, w_ref[...], preferred_element_type=jnp.float32)
    xl_ref[...] = xl
    al_ref[...] = jnp.sum(xl * as_ref[...], axis=-1, keepdims=True)
    ar_ref[...] = jnp.sum(xl * ad_ref[...], axis=-1, keepdims=True)


def _tc_merge(acc0, acc1, den0, den1, b, g, bt, w, a_s, a_d):
    hin, hout = w.shape
    return pl.pallas_call(
        _tc_merge_body,
        grid=(_GRID,),
        in_specs=[
            _row_spec(hin),
            _row_spec(hin),
            _row_spec(1),
            _row_spec(1),
            _full_spec((1, hin)),
            _full_spec((1, hin)),
            _full_spec((1, hin)),
            _full_spec((hin, hout)),
            _full_spec((1, hout)),
            _full_spec((1, hout)),
        ],
        out_specs=[
            _row_spec(hout),
            _row_spec(1),
            _row_spec(1),
        ],
        out_shape=[
            jax.ShapeDtypeStruct((N, hout), jnp.float32),
            jax.ShapeDtypeStruct((N, 1), jnp.float32),
            jax.ShapeDtypeStruct((N, 1), jnp.float32),
        ],
    )(acc0, acc1, den0, den1, b, g, bt, w, a_s, a_d)


def _tc_final_body(acc0_ref, acc1_ref, den0_ref, den1_ref, b_ref, out_ref):
    a = acc0_ref[...] + acc1_ref[...]
    d = den0_ref[...] + den1_ref[...]
    out_ref[...] = a / (d + 1e-16) + b_ref[...]


def _tc_final(acc0, acc1, den0, den1, b):
    h = acc0.shape[-1]
    return pl.pallas_call(
        _tc_final_body,
        grid=(_GRID,),
        in_specs=[
            _row_spec(h),
            _row_spec(h),
            _row_spec(1),
            _row_spec(1),
            _full_spec((1, h)),
        ],
        out_specs=_row_spec(h),
        out_shape=jax.ShapeDtypeStruct((N, h), jnp.float32),
    )(acc0, acc1, den0, den1, b)


def _run_sc(xl, al, ar, src, dst, zrows, zden):
    acc, den = _sc_layer(xl, al, ar, src, dst, zrows, zden)
    return (acc[0, :N], acc[1, :N],
            den[0, :N].reshape(N, 1), den[1, :N].reshape(N, 1))


def kernel(x, edge_index, W0, as0, ad0, b0, g0, bt0, W1, as1, ad1, b1, g1,
           bt1, W2, as2, ad2, b2, g2, bt2, Wf, asf, adf, bf):
    loop = jnp.arange(N, dtype=jnp.int32)
    src = jnp.concatenate([edge_index[0].astype(jnp.int32), loop])
    dst = jnp.concatenate([edge_index[1].astype(jnp.int32), loop])
    pad = EPAD - E_REAL
    src = jnp.pad(src, (0, pad)).reshape(NW, T)
    dst = jnp.pad(dst, (0, pad)).reshape(NW, T)

    zrows = jnp.zeros((RPT, HID), jnp.float32)
    zden = jnp.zeros((RPT,), jnp.float32)

    def flat(a):
        return a.reshape(N)

    xl, al, ar = _tc_in(x, W0, as0, ad0)
    a0, a1, d0, d1 = _run_sc(xl, flat(al), flat(ar), src, dst, zrows, zden)
    xl, al, ar = _tc_merge(a0, a1, d0, d1, b0.reshape(1, HID),
                           g0.reshape(1, HID), bt0.reshape(1, HID),
                           W1, as1, ad1)
    a0, a1, d0, d1 = _run_sc(xl, flat(al), flat(ar), src, dst, zrows, zden)
    xl, al, ar = _tc_merge(a0, a1, d0, d1, b1.reshape(1, HID),
                           g1.reshape(1, HID), bt1.reshape(1, HID),
                           W2, as2, ad2)
    a0, a1, d0, d1 = _run_sc(xl, flat(al), flat(ar), src, dst, zrows, zden)
    xl, al, ar = _tc_merge(a0, a1, d0, d1, b2.reshape(1, HID),
                           g2.reshape(1, HID), bt2.reshape(1, HID),
                           Wf, asf, adf)
    alf, arf = flat(al), flat(ar)
    xla = xl[:, :HID]
    xlb = xl[:, HID:]
    a0a, a1a, d0, d1 = _run_sc(xla, alf, arf, src, dst, zrows, zden)
    a0b, a1b, _, _ = _run_sc(xlb, alf, arf, src, dst, zrows, zden)
    outa = _tc_final(a0a, a1a, d0, d1, bf[:HID].reshape(1, HID))
    outb = _tc_final(a0b, a1b, d0, d1, bf[HID:].reshape(1, HID))
    return jnp.concatenate([outa, outb], axis=1)


# speedup vs baseline: 18.2224x; 1.0787x over previous
import functools

import jax
import jax.numpy as jnp
from jax import lax
from jax.experimental import pallas as pl
from jax.experimental.pallas import tpu as pltpu
from jax.experimental.pallas import tpu_sc as plsc

N = 10000
DIN = 128
HID = 64
DOUT = 128
NEG = 0.2
E_REAL = 330000

NC = 2
NS = 16
NW = NC * NS
T = 10752
EPAD = NW * T
NPAD = 10240
RPT = NPAD // NS
CH = 448
NCHUNK = T // CH
NBUF = 2


def _make_sc_layer():
    D = HID
    mesh = plsc.VectorSubcoreMesh(core_axis_name="c", subcore_axis_name="s")

    scratch = dict(
        acc_sp=pltpu.VMEM_SHARED((NPAD, D), jnp.float32),
        den_sp=pltpu.VMEM_SHARED((NPAD,), jnp.float32),
        src_v=pltpu.VMEM((T,), jnp.int32),
        dst_v=pltpu.VMEM((T,), jnp.int32),
        asem=pltpu.SemaphoreType.DMA,
        gsem=pltpu.SemaphoreType.DMA,
        dsem=pltpu.SemaphoreType.DMA,
        ssem=pltpu.SemaphoreType.DMA,
    )
    for b in range(NBUF):
        scratch[f"rows{b}"] = pltpu.VMEM((CH, D), jnp.float32)
        scratch[f"av{b}"] = pltpu.VMEM((CH,), jnp.float32)
        scratch[f"bv{b}"] = pltpu.VMEM((CH,), jnp.float32)
        scratch[f"exc{b}"] = pltpu.VMEM((CH,), jnp.float32)

    @functools.partial(
        pl.kernel,
        out_type=(
            jax.ShapeDtypeStruct((NC, NPAD, D), jnp.float32),
            jax.ShapeDtypeStruct((NC, NPAD), jnp.float32),
        ),
        mesh=mesh,
        compiler_params=pltpu.CompilerParams(
            use_tc_tiling_on_sc=False, needs_layout_passes=False),
        scratch_types=scratch,
    )
    def sc_layer(xl_hbm, al_hbm, ar_hbm, srcw_hbm, dstw_hbm, zrows_hbm,
                 zden_hbm, acc_hbm, den_hbm, **scr):
        cid = lax.axis_index("c")
        sid = lax.axis_index("s")
        wid = cid * NS + sid
        rbase = sid * RPT

        rows = [scr[f"rows{b}"] for b in range(NBUF)]
        av = [scr[f"av{b}"] for b in range(NBUF)]
        bv = [scr[f"bv{b}"] for b in range(NBUF)]
        exc = [scr[f"exc{b}"] for b in range(NBUF)]
        acc_sp = scr["acc_sp"]
        den_sp = scr["den_sp"]
        src_v, dst_v = scr["src_v"], scr["dst_v"]
        asem, gsem = scr["asem"], scr["gsem"]
        dsem, ssem = scr["dsem"], scr["ssem"]

        pltpu.sync_copy(zrows_hbm, acc_sp.at[pl.ds(rbase, RPT)])
        pltpu.sync_copy(zden_hbm, den_sp.at[pl.ds(rbase, RPT)])
        pltpu.sync_copy(srcw_hbm.at[wid], src_v)
        pltpu.sync_copy(dstw_hbm.at[wid], dst_v)
        plsc.subcore_barrier()

        iota16 = lax.iota(jnp.int32, 16)

        def ssl(c):
            return src_v.at[pl.ds(c * CH, CH)]

        def dsl(c):
            return dst_v.at[pl.ds(c * CH, CH)]

        def launch(c):
            b = c % NBUF
            ha = pltpu.async_copy(al_hbm.at[ssl(c)], av[b], asem)
            hb = pltpu.async_copy(ar_hbm.at[dsl(c)], bv[b], asem)
            hg = pltpu.async_copy(xl_hbm.at[ssl(c)], rows[b], gsem)
            return ha, hb, hg

        pend_d = [None] * NBUF
        pend_s = [None] * NBUF
        cur = launch(0)
        for c in range(NCHUNK):
            b = c % NBUF
            nxt = None
            if c + 1 < NCHUNK:
                nb = (c + 1) % NBUF
                if pend_d[nb] is not None:
                    pend_d[nb].wait()
                    pend_s[nb].wait()
                nxt = launch(c + 1)
            ha, hb, hg = cur
            ha.wait()
            hb.wait()
            base_c = wid * T + c * CH

            def exp_body(j, carry, b=b, base_c=base_c):
                off = j * 16
                t = av[b][pl.ds(off, 16)] + bv[b][pl.ds(off, 16)]
                e = jnp.where(t >= 0.0, t, t * NEG)
                ex = jnp.exp(e)
                gid = base_c + off + iota16
                ex = jnp.where(gid < E_REAL, ex, 0.0)
                exc[b][pl.ds(off, 16)] = ex
                return carry

            lax.fori_loop(0, CH // 16, exp_body, 0)
            hd = pltpu.async_copy(exc[b], den_sp.at[dsl(c)], dsem,
                                  add=True)
            hg.wait()

            def scale_body(jb, carry, b=b):
                exv = exc[b][pl.ds(jb * 16, 16)]
                for k in range(16):
                    e = exv[k]
                    row = jb * 16 + k
                    for g in range(D // 16):
                        sl = pl.ds(g * 16, 16)
                        rows[b][row, sl] = rows[b][row, sl] * e
                return carry

            lax.fori_loop(0, CH // 16, scale_body, 0)
            hs = pltpu.async_copy(rows[b], acc_sp.at[dsl(c)], ssem,
                                  add=True)
            pend_d[b] = hd
            pend_s[b] = hs
            cur = nxt
        for b in range(NBUF):
            if pend_d[b] is not None:
                pend_d[b].wait()
                pend_s[b].wait()

        plsc.subcore_barrier()
        pltpu.sync_copy(
            acc_sp.at[pl.ds(rbase, RPT)],
            acc_hbm.at[cid].at[pl.ds(rbase, RPT)])
        pltpu.sync_copy(
            den_sp.at[pl.ds(rbase, RPT)],
            den_hbm.at[cid].at[pl.ds(rbase, RPT)])

    return sc_layer


_sc_layer = _make_sc_layer()


_BN = 1000
_GRID = N // _BN


def _row_spec(width):
    return pl.BlockSpec((_BN, width), lambda i: (i, 0))


def _full_spec(shape):
    nd = len(shape)
    return pl.BlockSpec(shape, lambda i: (0,) * nd)


def _tc_in_body(x_ref, w_ref, as_ref, ad_ref, xl_ref, al_ref, ar_ref):
    xl = jnp.dot(x_ref[...], w_ref[...], preferred_element_type=jnp.float32)
    xl_ref[...] = xl
    al_ref[...] = jnp.sum(xl * as_ref[...], axis=-1, keepdims=True)
    ar_ref[...] = jnp.sum(xl * ad_ref[...], axis=-1, keepdims=True)


def _tc_in(x, w, a_s, a_d):
    din, h = w.shape
    return pl.pallas_call(
        _tc_in_body,
        grid=(_GRID,),
        in_specs=[
            _row_spec(din),
            _full_spec((din, h)),
            _full_spec((1, h)),
            _full_spec((1, h)),
        ],
        out_specs=[
            _row_spec(h),
            _row_spec(1),
            _row_spec(1),
        ],
        out_shape=[
            jax.ShapeDtypeStruct((N, h), jnp.float32),
            jax.ShapeDtypeStruct((N, 1), jnp.float32),
            jax.ShapeDtypeStruct((N, 1), jnp.float32),
        ],
    )(x, w, a_s, a_d)


def _tc_merge_body(acc0_ref, acc1_ref, den0_ref, den1_ref, b_ref, g_ref,
                   bt_ref, w_ref, as_ref, ad_ref, xl_ref, al_ref, ar_ref):
    a = acc0_ref[...] + acc1_ref[...]
    d = den0_ref[...] + den1_ref[...]
    h = a / (d + 1e-16) + b_ref[...]
    mu = jnp.mean(h, axis=-1, keepdims=True)
    var = jnp.mean((h - mu) ** 2, axis=-1, keepdims=True)
    h = (h - mu) / jnp.sqrt(var + 1e-5) * g_ref[...] + bt_ref[...]
    h = jnp.where(h > 0.0, h, jnp.exp(h) - 1.0)
    xl = jnp.dot(h, w_ref[...], preferred_element_type=jnp.float32)
    xl_ref[...] = xl
    al_ref[...] = jnp.sum(xl * as_ref[...], axis=-1, keepdims=True)
    ar_ref[...] = jnp.sum(xl * ad_ref[...], axis=-1, keepdims=True)


def _tc_merge(acc0, acc1, den0, den1, b, g, bt, w, a_s, a_d):
    hin, hout = w.shape
    return pl.pallas_call(
        _tc_merge_body,
        grid=(_GRID,),
        in_specs=[
            _row_spec(hin),
            _row_spec(hin),
            _row_spec(1),
            _row_spec(1),
            _full_spec((1, hin)),
            _full_spec((1, hin)),
            _full_spec((1, hin)),
            _full_spec((hin, hout)),
            _full_spec((1, hout)),
            _full_spec((1, hout)),
        ],
        out_specs=[
            _row_spec(hout),
            _row_spec(1),
            _row_spec(1),
        ],
        out_shape=[
            jax.ShapeDtypeStruct((N, hout), jnp.float32),
            jax.ShapeDtypeStruct((N, 1), jnp.float32),
            jax.ShapeDtypeStruct((N, 1), jnp.float32),
        ],
    )(acc0, acc1, den0, den1, b, g, bt, w, a_s, a_d)


def _tc_final_body(acc0_ref, acc1_ref, den0_ref, den1_ref, b_ref, out_ref):
    a = acc0_ref[...] + acc1_ref[...]
    d = den0_ref[...] + den1_ref[...]
    out_ref[...] = a / (d + 1e-16) + b_ref[...]


def _tc_final(acc0, acc1, den0, den1, b):
    h = acc0.shape[-1]
    return pl.pallas_call(
        _tc_final_body,
        grid=(_GRID,),
        in_specs=[
            _row_spec(h),
            _row_spec(h),
            _row_spec(1),
            _row_spec(1),
            _full_spec((1, h)),
        ],
        out_specs=_row_spec(h),
        out_shape=jax.ShapeDtypeStruct((N, h), jnp.float32),
    )(acc0, acc1, den0, den1, b)


def _run_sc(xl, al, ar, src, dst, zrows, zden):
    acc, den = _sc_layer(xl, al, ar, src, dst, zrows, zden)
    return (acc[0, :N], acc[1, :N],
            den[0, :N].reshape(N, 1), den[1, :N].reshape(N, 1))


def kernel(x, edge_index, W0, as0, ad0, b0, g0, bt0, W1, as1, ad1, b1, g1,
           bt1, W2, as2, ad2, b2, g2, bt2, Wf, asf, adf, bf):
    loop = jnp.arange(N, dtype=jnp.int32)
    src = jnp.concatenate([edge_index[0].astype(jnp.int32), loop])
    dst = jnp.concatenate([edge_index[1].astype(jnp.int32), loop])
    pad = EPAD - E_REAL
    src = jnp.pad(src, (0, pad)).reshape(NW, T)
    dst = jnp.pad(dst, (0, pad)).reshape(NW, T)

    zrows = jnp.zeros((RPT, HID), jnp.float32)
    zden = jnp.zeros((RPT,), jnp.float32)

    def flat(a):
        return a.reshape(N)

    xl, al, ar = _tc_in(x, W0, as0, ad0)
    a0, a1, d0, d1 = _run_sc(xl, flat(al), flat(ar), src, dst, zrows, zden)
    xl, al, ar = _tc_merge(a0, a1, d0, d1, b0.reshape(1, HID),
                           g0.reshape(1, HID), bt0.reshape(1, HID),
                           W1, as1, ad1)
    a0, a1, d0, d1 = _run_sc(xl, flat(al), flat(ar), src, dst, zrows, zden)
    xl, al, ar = _tc_merge(a0, a1, d0, d1, b1.reshape(1, HID),
                           g1.reshape(1, HID), bt1.reshape(1, HID),
                           W2, as2, ad2)
    a0, a1, d0, d1 = _run_sc(xl, flat(al), flat(ar), src, dst, zrows, zden)
    xl, al, ar = _tc_merge(a0, a1, d0, d1, b2.reshape(1, HID),
                           g2.reshape(1, HID), bt2.reshape(1, HID),
                           Wf, asf, adf)
    alf, arf = flat(al), flat(ar)
    xla = xl[:, :HID]
    xlb = xl[:, HID:]
    a0a, a1a, d0, d1 = _run_sc(xla, alf, arf, src, dst, zrows, zden)
    a0b, a1b, _, _ = _run_sc(xlb, alf, arf, src, dst, zrows, zden)
    outa = _tc_final(a0a, a1a, d0, d1, bf[:HID].reshape(1, HID))
    outb = _tc_final(a0b, a1b, d0, d1, bf[HID:].reshape(1, HID))
    return jnp.concatenate([outa, outb], axis=1)


# speedup vs baseline: 18.3688x; 1.0080x over previous
import functools

import jax
import jax.numpy as jnp
from jax import lax
from jax.experimental import pallas as pl
from jax.experimental.pallas import tpu as pltpu
from jax.experimental.pallas import tpu_sc as plsc

N = 10000
DIN = 128
HID = 64
DOUT = 128
NEG = 0.2
E_REAL = 330000

NC = 2
NS = 16
NW = NC * NS
T = 10752
EPAD = NW * T
NPAD = 10240
RPT = NPAD // NS
CH = 336
NCHUNK = T // CH
NBUF = 2
RBUF = 3


def _make_sc_layer():
    D = HID
    mesh = plsc.VectorSubcoreMesh(core_axis_name="c", subcore_axis_name="s")

    scratch = dict(
        acc_sp=pltpu.VMEM_SHARED((NPAD, D), jnp.float32),
        den_sp=pltpu.VMEM_SHARED((NPAD,), jnp.float32),
        src_v=pltpu.VMEM((T,), jnp.int32),
        dst_v=pltpu.VMEM((T,), jnp.int32),
        asem=pltpu.SemaphoreType.DMA,
        gsem=pltpu.SemaphoreType.DMA,
        dsem=pltpu.SemaphoreType.DMA,
        ssem=pltpu.SemaphoreType.DMA,
    )
    for b in range(RBUF):
        scratch[f"rows{b}"] = pltpu.VMEM((CH, D), jnp.float32)
    for b in range(NBUF):
        scratch[f"av{b}"] = pltpu.VMEM((CH,), jnp.float32)
        scratch[f"bv{b}"] = pltpu.VMEM((CH,), jnp.float32)
        scratch[f"exc{b}"] = pltpu.VMEM((CH,), jnp.float32)

    @functools.partial(
        pl.kernel,
        out_type=(
            jax.ShapeDtypeStruct((NC, NPAD, D), jnp.float32),
            jax.ShapeDtypeStruct((NC, NPAD), jnp.float32),
        ),
        mesh=mesh,
        compiler_params=pltpu.CompilerParams(
            use_tc_tiling_on_sc=False, needs_layout_passes=False),
        scratch_types=scratch,
    )
    def sc_layer(xl_hbm, al_hbm, ar_hbm, srcw_hbm, dstw_hbm, zrows_hbm,
                 zden_hbm, acc_hbm, den_hbm, **scr):
        cid = lax.axis_index("c")
        sid = lax.axis_index("s")
        wid = cid * NS + sid
        rbase = sid * RPT

        rows = [scr[f"rows{b}"] for b in range(RBUF)]
        av = [scr[f"av{b}"] for b in range(NBUF)]
        bv = [scr[f"bv{b}"] for b in range(NBUF)]
        exc = [scr[f"exc{b}"] for b in range(NBUF)]
        acc_sp = scr["acc_sp"]
        den_sp = scr["den_sp"]
        src_v, dst_v = scr["src_v"], scr["dst_v"]
        asem, gsem = scr["asem"], scr["gsem"]
        dsem, ssem = scr["dsem"], scr["ssem"]

        pltpu.sync_copy(zrows_hbm, acc_sp.at[pl.ds(rbase, RPT)])
        pltpu.sync_copy(zden_hbm, den_sp.at[pl.ds(rbase, RPT)])
        pltpu.sync_copy(srcw_hbm.at[wid], src_v)
        pltpu.sync_copy(dstw_hbm.at[wid], dst_v)
        plsc.subcore_barrier()

        iota16 = lax.iota(jnp.int32, 16)

        def ssl(c):
            return src_v.at[pl.ds(c * CH, CH)]

        def dsl(c):
            return dst_v.at[pl.ds(c * CH, CH)]

        def launch_ab(c):
            b = c % NBUF
            ha = pltpu.async_copy(al_hbm.at[ssl(c)], av[b], asem)
            hb = pltpu.async_copy(ar_hbm.at[dsl(c)], bv[b], asem)
            return ha, hb

        def launch_rows(c):
            return pltpu.async_copy(
                xl_hbm.at[ssl(c)], rows[c % RBUF], gsem)

        pend_d = [None] * NBUF
        pend_s = [None] * RBUF
        cur_ab = launch_ab(0)
        cur_g = launch_rows(0)
        nxt_ab = nxt_g = None
        for c in range(NCHUNK):
            b = c % NBUF
            r = c % RBUF
            if c + 1 < NCHUNK:
                nb = (c + 1) % NBUF
                if pend_d[nb] is not None:
                    pend_d[nb].wait()
                nxt_ab = launch_ab(c + 1)
            ha, hb = cur_ab
            ha.wait()
            hb.wait()
            base_c = wid * T + c * CH

            def exp_body(j, carry, b=b, base_c=base_c):
                off = j * 16
                t = av[b][pl.ds(off, 16)] + bv[b][pl.ds(off, 16)]
                e = jnp.where(t >= 0.0, t, t * NEG)
                ex = jnp.exp(e)
                gid = base_c + off + iota16
                ex = jnp.where(gid < E_REAL, ex, 0.0)
                exc[b][pl.ds(off, 16)] = ex
                return carry

            lax.fori_loop(0, CH // 16, exp_body, 0)
            hd = pltpu.async_copy(exc[b], den_sp.at[dsl(c)], dsem,
                                  add=True)
            if c + 1 < NCHUNK:
                nr = (c + 1) % RBUF
                if pend_s[nr] is not None:
                    pend_s[nr].wait()
                nxt_g = launch_rows(c + 1)
            cur_g.wait()

            def scale_body(jb, carry, b=b, r=r):
                exv = exc[b][pl.ds(jb * 16, 16)]
                for k in range(16):
                    e = exv[k]
                    row = jb * 16 + k
                    for g in range(D // 16):
                        sl = pl.ds(g * 16, 16)
                        rows[r][row, sl] = rows[r][row, sl] * e
                return carry

            lax.fori_loop(0, CH // 16, scale_body, 0)
            hs = pltpu.async_copy(rows[r], acc_sp.at[dsl(c)], ssem,
                                  add=True)
            pend_d[b] = hd
            pend_s[r] = hs
            cur_ab = nxt_ab
            cur_g = nxt_g
        for b in range(NBUF):
            if pend_d[b] is not None:
                pend_d[b].wait()
        for r in range(RBUF):
            if pend_s[r] is not None:
                pend_s[r].wait()

        plsc.subcore_barrier()
        pltpu.sync_copy(
            acc_sp.at[pl.ds(rbase, RPT)],
            acc_hbm.at[cid].at[pl.ds(rbase, RPT)])
        pltpu.sync_copy(
            den_sp.at[pl.ds(rbase, RPT)],
            den_hbm.at[cid].at[pl.ds(rbase, RPT)])

    return sc_layer


_sc_layer = _make_sc_layer()


_BN = 1000
_GRID = N // _BN


def _row_spec(width):
    return pl.BlockSpec((_BN, width), lambda i: (i, 0))


def _full_spec(shape):
    nd = len(shape)
    return pl.BlockSpec(shape, lambda i: (0,) * nd)


def _tc_in_body(x_ref, w_ref, as_ref, ad_ref, xl_ref, al_ref, ar_ref):
    xl = jnp.dot(x_ref[...], w_ref[...], preferred_element_type=jnp.float32)
    xl_ref[...] = xl
    al_ref[...] = jnp.sum(xl * as_ref[...], axis=-1, keepdims=True)
    ar_ref[...] = jnp.sum(xl * ad_ref[...], axis=-1, keepdims=True)


def _tc_in(x, w, a_s, a_d):
    din, h = w.shape
    return pl.pallas_call(
        _tc_in_body,
        grid=(_GRID,),
        in_specs=[
            _row_spec(din),
            _full_spec((din, h)),
            _full_spec((1, h)),
            _full_spec((1, h)),
        ],
        out_specs=[
            _row_spec(h),
            _row_spec(1),
            _row_spec(1),
        ],
        out_shape=[
            jax.ShapeDtypeStruct((N, h), jnp.float32),
            jax.ShapeDtypeStruct((N, 1), jnp.float32),
            jax.ShapeDtypeStruct((N, 1), jnp.float32),
        ],
    )(x, w, a_s, a_d)


def _tc_merge_body(acc0_ref, acc1_ref, den0_ref, den1_ref, b_ref, g_ref,
                   bt_ref, w_ref, as_ref, ad_ref, xl_ref, al_ref, ar_ref):
    a = acc0_ref[...] + acc1_ref[...]
    d = den0_ref[...] + den1_ref[...]
    h = a / (d + 1e-16) + b_ref[...]
    mu = jnp.mean(h, axis=-1, keepdims=True)
    var = jnp.mean((h - mu) ** 2, axis=-1, keepdims=True)
    h = (h - mu) / jnp.sqrt(var + 1e-5) * g_ref[...] + bt_ref[...]
    h = jnp.where(h > 0.0, h, jnp.exp(h) - 1.0)
    xl = jnp.dot(h, w_ref[...], preferred_element_type=jnp.float32)
    xl_ref[...] = xl
    al_ref[...] = jnp.sum(xl * as_ref[...], axis=-1, keepdims=True)
    ar_ref[...] = jnp.sum(xl * ad_ref[...], axis=-1, keepdims=True)


def _tc_merge(acc0, acc1, den0, den1, b, g, bt, w, a_s, a_d):
    hin, hout = w.shape
    return pl.pallas_call(
        _tc_merge_body,
        grid=(_GRID,),
        in_specs=[
            _row_spec(hin),
            _row_spec(hin),
            _row_spec(1),
            _row_spec(1),
            _full_spec((1, hin)),
            _full_spec((1, hin)),
            _full_spec((1, hin)),
            _full_spec((hin, hout)),
            _full_spec((1, hout)),
            _full_spec((1, hout)),
        ],
        out_specs=[
            _row_spec(hout),
            _row_spec(1),
            _row_spec(1),
        ],
        out_shape=[
            jax.ShapeDtypeStruct((N, hout), jnp.float32),
            jax.ShapeDtypeStruct((N, 1), jnp.float32),
            jax.ShapeDtypeStruct((N, 1), jnp.float32),
        ],
    )(acc0, acc1, den0, den1, b, g, bt, w, a_s, a_d)


def _tc_final_body(acc0_ref, acc1_ref, den0_ref, den1_ref, b_ref, out_ref):
    a = acc0_ref[...] + acc1_ref[...]
    d = den0_ref[...] + den1_ref[...]
    out_ref[...] = a / (d + 1e-16) + b_ref[...]


def _tc_final(acc0, acc1, den0, den1, b):
    h = acc0.shape[-1]
    return pl.pallas_call(
        _tc_final_body,
        grid=(_GRID,),
        in_specs=[
            _row_spec(h),
            _row_spec(h),
            _row_spec(1),
            _row_spec(1),
            _full_spec((1, h)),
        ],
        out_specs=_row_spec(h),
        out_shape=jax.ShapeDtypeStruct((N, h), jnp.float32),
    )(acc0, acc1, den0, den1, b)


def _run_sc(xl, al, ar, src, dst, zrows, zden):
    acc, den = _sc_layer(xl, al, ar, src, dst, zrows, zden)
    return (acc[0, :N], acc[1, :N],
            den[0, :N].reshape(N, 1), den[1, :N].reshape(N, 1))


def kernel(x, edge_index, W0, as0, ad0, b0, g0, bt0, W1, as1, ad1, b1, g1,
           bt1, W2, as2, ad2, b2, g2, bt2, Wf, asf, adf, bf):
    loop = jnp.arange(N, dtype=jnp.int32)
    src = jnp.concatenate([edge_index[0].astype(jnp.int32), loop])
    dst = jnp.concatenate([edge_index[1].astype(jnp.int32), loop])
    pad = EPAD - E_REAL
    src = jnp.pad(src, (0, pad)).reshape(NW, T)
    dst = jnp.pad(dst, (0, pad)).reshape(NW, T)

    zrows = jnp.zeros((RPT, HID), jnp.float32)
    zden = jnp.zeros((RPT,), jnp.float32)

    def flat(a):
        return a.reshape(N)

    xl, al, ar = _tc_in(x, W0, as0, ad0)
    a0, a1, d0, d1 = _run_sc(xl, flat(al), flat(ar), src, dst, zrows, zden)
    xl, al, ar = _tc_merge(a0, a1, d0, d1, b0.reshape(1, HID),
                           g0.reshape(1, HID), bt0.reshape(1, HID),
                           W1, as1, ad1)
    a0, a1, d0, d1 = _run_sc(xl, flat(al), flat(ar), src, dst, zrows, zden)
    xl, al, ar = _tc_merge(a0, a1, d0, d1, b1.reshape(1, HID),
                           g1.reshape(1, HID), bt1.reshape(1, HID),
                           W2, as2, ad2)
    a0, a1, d0, d1 = _run_sc(xl, flat(al), flat(ar), src, dst, zrows, zden)
    xl, al, ar = _tc_merge(a0, a1, d0, d1, b2.reshape(1, HID),
                           g2.reshape(1, HID), bt2.reshape(1, HID),
                           Wf, asf, adf)
    alf, arf = flat(al), flat(ar)
    xla = xl[:, :HID]
    xlb = xl[:, HID:]
    a0a, a1a, d0, d1 = _run_sc(xla, alf, arf, src, dst, zrows, zden)
    a0b, a1b, _, _ = _run_sc(xlb, alf, arf, src, dst, zrows, zden)
    outa = _tc_final(a0a, a1a, d0, d1, bf[:HID].reshape(1, HID))
    outb = _tc_final(a0b, a1b, d0, d1, bf[HID:].reshape(1, HID))
    return jnp.concatenate([outa, outb], axis=1)
